# Initial kernel scaffold; baseline (speedup 1.0000x reference)
import jax
import jax.numpy as jnp
from jax.experimental import pallas as pl


def kernel(x1, edge_index1, batch1, x2, edge_index2, batch2, W_in, b_in, W_self, b_self, W_nbr, b_nbr, Wp1, bp1, Wp2, bp2):
    raise NotImplementedError("write your pallas kernel here")


# speedup vs baseline: 2.9802x; 2.9802x over previous
import functools

import jax
import jax.numpy as jnp
from jax import lax
from jax.experimental import pallas as pl
from jax.experimental.pallas import tpu as pltpu
from jax.experimental.pallas import tpu_sc as plsc

N = 10000
E = 160000
D = 256
H = 512
L = 3
G = 64

NP = 10240
HC = 4
CW = H // HC
NT = 16
EP = E // NT
B = 80
NB = EP // B
RT = NP // NT
R = 512
NR = NP // R


def _segsum_body(hall, edges, msg, dst2d, gidx2d, rows, zbuf, acc, sem):
    g = lax.axis_index("c")
    s = lax.axis_index("s")

    pltpu.sync_copy(edges.at[g, 0, s], gidx2d)
    pltpu.sync_copy(edges.at[g, 1, s], dst2d)

    def _zb(i, _):
        zbuf[i // 8, pl.ds((i % 8) * 16, 16)] = jnp.zeros((16,), jnp.float32)
        return 0
    lax.fori_loop(0, 32 * 8, _zb, 0)

    for c in range(HC):
        off = g * HC * NP if c == 0 else NP

        def _gi(i, _):
            j = i // (B // 16)
            k = i % (B // 16)
            gidx2d[j, pl.ds(k * 16, 16)] = gidx2d[j, pl.ds(k * 16, 16)] + off
            return 0
        lax.fori_loop(0, NB * (B // 16), _gi, 0)

        for k in range(RT // 32):
            pltpu.sync_copy(zbuf, acc.at[pl.ds(s * RT + k * 32, 32)])
        plsc.subcore_barrier()

        def _blk(j, _):
            pltpu.async_copy(hall.at[gidx2d.at[j]], rows, sem).wait()
            pltpu.sync_copy(rows, acc.at[dst2d.at[j]], add=True)
            return 0
        lax.fori_loop(0, NB, _blk, 0)
        plsc.subcore_barrier()

        pltpu.sync_copy(acc.at[pl.ds(s * RT, RT)],
                        msg.at[g, c, pl.ds(s * RT, RT)])
        plsc.subcore_barrier()


@functools.lru_cache(maxsize=1)
def _make_segsum():
    return pl.kernel(
        _segsum_body,
        out_type=jax.ShapeDtypeStruct((2, HC, NP, CW), jnp.float32),
        mesh=plsc.VectorSubcoreMesh(core_axis_name="c", subcore_axis_name="s"),
        scratch_types=[
            pltpu.VMEM((NB, B), jnp.int32),
            pltpu.VMEM((NB, B), jnp.int32),
            pltpu.VMEM((B, CW), jnp.float32),
            pltpu.VMEM((32, CW), jnp.float32),
            pltpu.VMEM_SHARED((NP, CW), jnp.float32),
            pltpu.SemaphoreType.DMA,
        ],
    )


def _segsum(hall, edges):
    return _make_segsum()(hall, edges)


def _proj_body(x_ref, w_ref, b_ref, out_ref):
    res = jnp.dot(x_ref[0], w_ref[...], preferred_element_type=jnp.float32)
    res = jnp.maximum(res + b_ref[...], 0.0)
    for c in range(HC):
        out_ref[0, c] = res[:, c * CW:(c + 1) * CW]


def _proj(xs, w_in, b_in):
    return pl.pallas_call(
        _proj_body,
        grid=(2, NR),
        in_specs=[
            pl.BlockSpec((1, R, D), lambda g, r: (g, r, 0)),
            pl.BlockSpec((D, H), lambda g, r: (0, 0)),
            pl.BlockSpec((1, H), lambda g, r: (0, 0)),
        ],
        out_specs=pl.BlockSpec((1, HC, R, CW), lambda g, r: (g, 0, r, 0)),
        out_shape=jax.ShapeDtypeStruct((2, HC, NP, CW), jnp.float32),
        compiler_params=pltpu.CompilerParams(
            dimension_semantics=("arbitrary", "arbitrary")),
    )(xs, w_in, b_in)


def _layer_body(h_ref, m_ref, ws_ref, wn_ref, b_ref, out_ref):
    acc = jnp.zeros((R, H), jnp.float32)
    for kc in range(HC):
        acc += jnp.dot(h_ref[0, kc], ws_ref[kc], preferred_element_type=jnp.float32)
        acc += jnp.dot(m_ref[0, kc], wn_ref[kc], preferred_element_type=jnp.float32)
    res = jnp.maximum(acc + b_ref[...], 0.0)
    for c in range(HC):
        out_ref[0, c] = res[:, c * CW:(c + 1) * CW]


def _layer(h4, m4, ws4, wn4, b):
    return pl.pallas_call(
        _layer_body,
        grid=(2, NR),
        in_specs=[
            pl.BlockSpec((1, HC, R, CW), lambda g, r: (g, 0, r, 0)),
            pl.BlockSpec((1, HC, R, CW), lambda g, r: (g, 0, r, 0)),
            pl.BlockSpec((HC, CW, H), lambda g, r: (0, 0, 0)),
            pl.BlockSpec((HC, CW, H), lambda g, r: (0, 0, 0)),
            pl.BlockSpec((1, H), lambda g, r: (0, 0)),
        ],
        out_specs=pl.BlockSpec((1, HC, R, CW), lambda g, r: (g, 0, r, 0)),
        out_shape=jax.ShapeDtypeStruct((2, HC, NP, CW), jnp.float32),
        compiler_params=pltpu.CompilerParams(
            dimension_semantics=("arbitrary", "arbitrary")),
    )(h4, m4, ws4, wn4, b)


def _pool_body(h_ref, m_ref, ws_ref, wn_ref, b_ref, batch_ref, emb_ref):
    r = pl.program_id(1)
    acc = jnp.zeros((R, H), jnp.float32)
    for kc in range(HC):
        acc += jnp.dot(h_ref[0, kc], ws_ref[kc], preferred_element_type=jnp.float32)
        acc += jnp.dot(m_ref[0, kc], wn_ref[kc], preferred_element_type=jnp.float32)
    res = jnp.maximum(acc + b_ref[...], 0.0)
    bvec = batch_ref[0, 0]
    oh = (lax.broadcasted_iota(jnp.int32, (G, R), 0) == bvec[None, :]
          ).astype(jnp.float32)
    contrib = jnp.dot(oh, res, preferred_element_type=jnp.float32)

    @pl.when(r == 0)
    def _():
        emb_ref[0] = jnp.zeros((G, H), jnp.float32)
    emb_ref[0] += contrib


def _pool(h4, m4, ws4, wn4, b, batch_r):
    return pl.pallas_call(
        _pool_body,
        grid=(2, NR),
        in_specs=[
            pl.BlockSpec((1, HC, R, CW), lambda g, r: (g, 0, r, 0)),
            pl.BlockSpec((1, HC, R, CW), lambda g, r: (g, 0, r, 0)),
            pl.BlockSpec((HC, CW, H), lambda g, r: (0, 0, 0)),
            pl.BlockSpec((HC, CW, H), lambda g, r: (0, 0, 0)),
            pl.BlockSpec((1, H), lambda g, r: (0, 0)),
            pl.BlockSpec((1, 1, R), lambda g, r: (g * NR + r, 0, 0)),
        ],
        out_specs=pl.BlockSpec((1, G, H), lambda g, r: (g, 0, 0)),
        out_shape=jax.ShapeDtypeStruct((2, G, H), jnp.float32),
        compiler_params=pltpu.CompilerParams(
            dimension_semantics=("arbitrary", "arbitrary")),
    )(h4, m4, ws4, wn4, b, batch_r)


def _head_body(emb_ref, w1_ref, b1_ref, w2_ref, b2_ref, out_ref):
    pair = jnp.abs(emb_ref[0] - emb_ref[1])
    hmid = jnp.dot(pair, w1_ref[...], preferred_element_type=jnp.float32)
    hmid = jnp.maximum(hmid + b1_ref[...], 0.0)
    z = jnp.sum(hmid * w2_ref[...], axis=1) + b2_ref[0, 0]
    out_ref[0] = 1.0 / (1.0 + jnp.exp(-z))


def _head(emb, w1, b1, w2row, b2):
    return pl.pallas_call(
        _head_body,
        in_specs=[
            pl.BlockSpec((2, G, H), lambda: (0, 0, 0)),
            pl.BlockSpec((H, 2 * H), lambda: (0, 0)),
            pl.BlockSpec((1, 2 * H), lambda: (0, 0)),
            pl.BlockSpec((1, 2 * H), lambda: (0, 0)),
            pl.BlockSpec(memory_space=pltpu.SMEM),
        ],
        out_specs=pl.BlockSpec((1, G), lambda: (0, 0)),
        out_shape=jax.ShapeDtypeStruct((1, G), jnp.float32),
    )(emb, w1, b1, w2row, b2)


def kernel(x1, edge_index1, batch1, x2, edge_index2, batch2,
           W_in, b_in, W_self, b_self, W_nbr, b_nbr, Wp1, bp1, Wp2, bp2):
    f32 = jnp.float32
    pad = NP - N
    xs = jnp.stack([jnp.pad(x1, ((0, pad), (0, 0))),
                    jnp.pad(x2, ((0, pad), (0, 0)))])
    edges = jnp.stack([edge_index1, edge_index2]).reshape(2, 2, NT, NB, B)
    batch_r = jnp.stack([
        jnp.pad(batch1, (0, pad), constant_values=G),
        jnp.pad(batch2, (0, pad), constant_values=G),
    ]).reshape(2 * NR, 1, R)
    ws4 = W_self.reshape(L, HC, CW, H)
    wn4 = W_nbr.reshape(L, HC, CW, H)
    bl = (b_self + b_nbr).reshape(L, 1, H)
    b_in2 = b_in.reshape(1, H)
    bp1r = bp1.reshape(1, 2 * H)
    wp2r = Wp2.reshape(1, 2 * H)
    bp2r = bp2.reshape(1, 1).astype(f32)

    h4 = _proj(xs, W_in, b_in2)
    for l in range(L):
        hall = h4.reshape(2 * HC * NP, CW)
        m4 = _segsum(hall, edges)
        if l < L - 1:
            h4 = _layer(h4, m4, ws4[l], wn4[l], bl[l])
        else:
            emb = _pool(h4, m4, ws4[l], wn4[l], bl[l], batch_r)
    out = _head(emb, Wp1, bp1r, wp2r, bp2r)
    return out.reshape(G, 1)
